# baseline (speedup 1.0000x reference)
import functools
import math

import jax
import jax.numpy as jnp
from jax import lax
from jax.experimental import pallas as pl
from jax.experimental.pallas import tpu as pltpu
from jax.experimental.pallas import tpu_sc as plsc

VOCAB = 1000000
EMBED = 64
MAXLEN = 200
PAD = 0
B = 4096
L = 200

NC = 2
NS = 16
NW = NC * NS

LANES = 16
BBLK = 128
LT = L // 8
DEPTH = 6
SCALE = math.sqrt(float(EMBED))


def r_prev(l):
    return lax.rem(l + (DEPTH - 1), DEPTH)


def _make_pos_embed(max_length, embed_size):
    t = jnp.arange(1, max_length + 1, dtype=jnp.float32)
    omega = jnp.arange(1, embed_size // 2 + 1, dtype=jnp.float32) / embed_size
    wt = t[:, None] * jnp.power(10000.0, -omega)[None, :]
    pos = jnp.zeros((max_length, embed_size), dtype=jnp.float32)
    pos = pos.at[:, 0::2].set(jnp.sin(wt))
    pos = pos.at[:, 1::2].set(jnp.cos(wt))
    return pos


def _body(x4_hbm, table_hbm, pos_hbm, out5_hbm, idxw, g, o, pos_v,
          sem_g, sem_o):
    wid = lax.axis_index("s") * NC + lax.axis_index("c")

    pltpu.sync_copy(x4_hbm.at[:, wid], idxw)
    pltpu.sync_copy(pos_hbm, pos_v)

    def fire(l, r):
        pltpu.async_copy(table_hbm.at[idxw.at[l // 8, l % 8]], g.at[r],
                         sem_g)

    for lp in range(DEPTH - 1):
        fire(lp, lp)

    def l_body(l, carry):
        lt = l // 8
        lr = l % 8
        r = lax.rem(l, DEPTH)
        ro = lax.rem(l, 2)

        @pl.when(l < L - (DEPTH - 1))
        def _():
            fire(l + (DEPTH - 1), r_prev(l))

        pltpu.make_async_copy(table_hbm.at[pl.ds(0, BBLK)], g.at[r],
                              sem_g).wait()

        @pl.when(l >= 2)
        def _():
            pltpu.make_async_copy(o.at[ro], out5_hbm.at[0, :, 0],
                                  sem_o).wait()

        sv = []
        for j in range(BBLK // LANES):
            iv = idxw[lt, lr, pl.ds(j * LANES, LANES)]
            sv.append(jnp.where(iv == PAD, jnp.float32(0.0),
                                jnp.float32(SCALE)))

        iot = lax.iota(jnp.int32, LANES)
        rv = lax.broadcast(r, (LANES,))
        rov = lax.broadcast(ro, (LANES,))
        rows_j = [iot + (j * LANES) for j in range(BBLK // LANES)]

        lv = lax.broadcast(l, (LANES,))

        def e_body(e, carry2):
            cvec = lax.bitwise_and(iot + e, jnp.int32(EMBED - 1))
            etv = lax.shift_right_logical(cvec, 3)
            erv = lax.bitwise_and(cvec, jnp.int32(7))
            pe = plsc.load_gather(pos_v, [lv, cvec])
            gvs = [plsc.load_gather(g, [rv, rows_j[j], cvec])
                   for j in range(BBLK // LANES)]
            vals = [gvs[j] * sv[j] + pe for j in range(BBLK // LANES)]
            for j in range(BBLK // LANES):
                plsc.store_scatter(o, [rov, etv, erv, rows_j[j]], vals[j])
            return carry2

        lax.fori_loop(0, EMBED, e_body, 0, unroll=2)

        pltpu.async_copy(o.at[ro], out5_hbm.at[l, :, wid], sem_o)
        return carry

    lax.fori_loop(0, L, l_body, 0)

    for _ in range(2):
        pltpu.make_async_copy(o.at[0], out5_hbm.at[0, :, 0], sem_o).wait()


@functools.lru_cache(maxsize=None)
def _emb_call():
    return functools.partial(
        pl.kernel,
        out_type=jax.ShapeDtypeStruct((L, EMBED // 8, B // BBLK, 8, BBLK),
                                      jnp.float32),
        mesh=plsc.VectorSubcoreMesh(
            core_axis_name="c", subcore_axis_name="s",
            num_cores=NC, num_subcores=NS),
        scratch_types=[
            pltpu.VMEM((LT, 8, BBLK), jnp.int32),
            pltpu.VMEM((DEPTH, BBLK, EMBED), jnp.float32),
            pltpu.VMEM((2, EMBED // 8, 8, BBLK), jnp.float32),
            pltpu.VMEM((L, EMBED), jnp.float32),
            pltpu.SemaphoreType.DMA,
            pltpu.SemaphoreType.DMA,
        ],
        compiler_params=pltpu.CompilerParams(use_tc_tiling_on_sc=False,
                                             needs_layout_passes=False),
    )(_body)


def kernel(x, table):
    x4 = (x.astype(jnp.int32)
          .reshape(B // BBLK, BBLK, LT, 8)
          .transpose(2, 0, 3, 1))
    pos = _make_pos_embed(MAXLEN, EMBED)[:L]
    out5 = _emb_call()(x4, table, pos)
    return out5.transpose(2, 4, 0, 1, 3).reshape(B, L, EMBED)


# speedup vs baseline: 1.3588x; 1.3588x over previous
import functools
import math

import jax
import jax.numpy as jnp
from jax import lax
from jax.experimental import pallas as pl
from jax.experimental.pallas import tpu as pltpu
from jax.experimental.pallas import tpu_sc as plsc

VOCAB = 1000000
EMBED = 64
MAXLEN = 200
PAD = 0
B = 4096
L = 200

NC = 2
NS = 16
NW = NC * NS

LANES = 16
BBLK = 128
LT = L // 8
DEPTH = 6
SCALE = math.sqrt(float(EMBED))


def r_prev(l):
    return lax.rem(l + (DEPTH - 1), DEPTH)


def _make_pos_embed(max_length, embed_size):
    t = jnp.arange(1, max_length + 1, dtype=jnp.float32)
    omega = jnp.arange(1, embed_size // 2 + 1, dtype=jnp.float32) / embed_size
    wt = t[:, None] * jnp.power(10000.0, -omega)[None, :]
    pos = jnp.zeros((max_length, embed_size), dtype=jnp.float32)
    pos = pos.at[:, 0::2].set(jnp.sin(wt))
    pos = pos.at[:, 1::2].set(jnp.cos(wt))
    return pos


def _body(x4_hbm, table_hbm, posb_hbm, out5_hbm, idxw, g, o, pb,
          sem_g, sem_p, sem_o):
    wid = lax.axis_index("s") * NC + lax.axis_index("c")

    pltpu.sync_copy(x4_hbm.at[:, wid], idxw)

    def fire(l, r):
        pltpu.async_copy(table_hbm.at[idxw.at[l // 8, l % 8]], g.at[r],
                         sem_g)
        pltpu.async_copy(posb_hbm.at[l], pb.at[r], sem_p)

    for lp in range(DEPTH - 1):
        fire(lp, lp)

    def l_body(l, carry):
        lt = l // 8
        lr = l % 8
        r = lax.rem(l, DEPTH)
        ro = lax.rem(l, 2)

        @pl.when(l < L - (DEPTH - 1))
        def _():
            fire(l + (DEPTH - 1), r_prev(l))

        pltpu.make_async_copy(table_hbm.at[pl.ds(0, BBLK)], g.at[r],
                              sem_g).wait()
        pltpu.make_async_copy(posb_hbm.at[0], pb.at[r], sem_p).wait()

        @pl.when(l >= 2)
        def _():
            pltpu.make_async_copy(o.at[ro], out5_hbm.at[0, :, 0],
                                  sem_o).wait()

        sv = []
        for j in range(BBLK // LANES):
            iv = idxw[lt, lr, pl.ds(j * LANES, LANES)]
            sv.append(jnp.where(iv == PAD, jnp.float32(0.0),
                                jnp.float32(SCALE)))

        iot = lax.iota(jnp.int32, LANES)
        rv = lax.broadcast(r, (LANES,))
        rov = lax.broadcast(ro, (LANES,))
        rows_j = [iot + (j * LANES) for j in range(BBLK // LANES)]

        def e_body(e, carry2):
            cvec = lax.bitwise_and(iot + e, jnp.int32(EMBED - 1))
            etv = lax.shift_right_logical(cvec, 3)
            erv = lax.bitwise_and(cvec, jnp.int32(7))
            pe = pb[r, e]
            gvs = [plsc.load_gather(g, [rv, rows_j[j], cvec])
                   for j in range(BBLK // LANES)]
            vals = [gvs[j] * sv[j] + pe for j in range(BBLK // LANES)]
            for j in range(BBLK // LANES):
                plsc.store_scatter(o, [rov, etv, erv, rows_j[j]], vals[j])
            return carry2

        lax.fori_loop(0, EMBED, e_body, 0, unroll=2)

        pltpu.async_copy(o.at[ro], out5_hbm.at[l, :, wid], sem_o)
        return carry

    lax.fori_loop(0, L, l_body, 0)

    for _ in range(2):
        pltpu.make_async_copy(o.at[0], out5_hbm.at[0, :, 0], sem_o).wait()


@functools.lru_cache(maxsize=None)
def _emb_call():
    return functools.partial(
        pl.kernel,
        out_type=jax.ShapeDtypeStruct((L, EMBED // 8, B // BBLK, 8, BBLK),
                                      jnp.float32),
        mesh=plsc.VectorSubcoreMesh(
            core_axis_name="c", subcore_axis_name="s",
            num_cores=NC, num_subcores=NS),
        scratch_types=[
            pltpu.VMEM((LT, 8, BBLK), jnp.int32),
            pltpu.VMEM((DEPTH, BBLK, EMBED), jnp.float32),
            pltpu.VMEM((2, EMBED // 8, 8, BBLK), jnp.float32),
            pltpu.VMEM((DEPTH, EMBED, LANES), jnp.float32),
            pltpu.SemaphoreType.DMA,
            pltpu.SemaphoreType.DMA,
            pltpu.SemaphoreType.DMA,
        ],
        compiler_params=pltpu.CompilerParams(use_tc_tiling_on_sc=False,
                                             needs_layout_passes=False),
    )(_body)


def kernel(x, table):
    x4 = (x.astype(jnp.int32)
          .reshape(B // BBLK, BBLK, LT, 8)
          .transpose(2, 0, 3, 1))
    pos = _make_pos_embed(MAXLEN, EMBED)[:L]
    ecol = (jnp.arange(EMBED)[:, None] + jnp.arange(LANES)[None, :]) % EMBED
    posb = pos[:, ecol]
    out5 = _emb_call()(x4, table, posb)
    return out5.transpose(2, 4, 0, 1, 3).reshape(B, L, EMBED)


# speedup vs baseline: 1.3655x; 1.0049x over previous
import functools
import math

import jax
import jax.numpy as jnp
from jax import lax
from jax.experimental import pallas as pl
from jax.experimental.pallas import tpu as pltpu
from jax.experimental.pallas import tpu_sc as plsc

VOCAB = 1000000
EMBED = 64
MAXLEN = 200
PAD = 0
B = 4096
L = 200

NC = 2
NS = 16
NW = NC * NS

LANES = 16
BBLK = 128
LT = L // 8
DEPTH = 8
SCALE = math.sqrt(float(EMBED))


def r_prev(l):
    return lax.rem(l + (DEPTH - 1), DEPTH)


def _make_pos_embed(max_length, embed_size):
    t = jnp.arange(1, max_length + 1, dtype=jnp.float32)
    omega = jnp.arange(1, embed_size // 2 + 1, dtype=jnp.float32) / embed_size
    wt = t[:, None] * jnp.power(10000.0, -omega)[None, :]
    pos = jnp.zeros((max_length, embed_size), dtype=jnp.float32)
    pos = pos.at[:, 0::2].set(jnp.sin(wt))
    pos = pos.at[:, 1::2].set(jnp.cos(wt))
    return pos


def _body(x4_hbm, table_hbm, posb_hbm, out5_hbm, idxw, g, o, pb,
          sem_g, sem_p, sem_o):
    wid = lax.axis_index("s") * NC + lax.axis_index("c")

    pltpu.sync_copy(x4_hbm.at[:, wid], idxw)

    def fire(l, r):
        pltpu.async_copy(table_hbm.at[idxw.at[l // 8, l % 8]], g.at[r],
                         sem_g)
        pltpu.async_copy(posb_hbm.at[l], pb.at[r], sem_p)

    for lp in range(DEPTH - 1):
        fire(lp, lp)

    def l_body(l, carry):
        lt = l // 8
        lr = l % 8
        r = lax.rem(l, DEPTH)
        ro = lax.rem(l, 2)

        @pl.when(l < L - (DEPTH - 1))
        def _():
            fire(l + (DEPTH - 1), r_prev(l))

        pltpu.make_async_copy(table_hbm.at[pl.ds(0, BBLK)], g.at[r],
                              sem_g).wait()
        pltpu.make_async_copy(posb_hbm.at[0], pb.at[r], sem_p).wait()

        @pl.when(l >= 2)
        def _():
            pltpu.make_async_copy(o.at[ro], out5_hbm.at[0, :, 0],
                                  sem_o).wait()

        sv = []
        for j in range(BBLK // LANES):
            iv = idxw[lt, lr, pl.ds(j * LANES, LANES)]
            sv.append(jnp.where(iv == PAD, jnp.float32(0.0),
                                jnp.float32(SCALE)))

        iot = lax.iota(jnp.int32, LANES)
        rv = lax.broadcast(r, (LANES,))
        rov = lax.broadcast(ro, (LANES,))
        rows_j = [iot + (j * LANES) for j in range(BBLK // LANES)]

        def e_body(e, carry2):
            cvec = lax.bitwise_and(iot + e, jnp.int32(EMBED - 1))
            etv = lax.shift_right_logical(cvec, 3)
            erv = lax.bitwise_and(cvec, jnp.int32(7))
            pe = pb[r, e]
            gvs = [plsc.load_gather(g, [rv, rows_j[j], cvec])
                   for j in range(BBLK // LANES)]
            vals = [gvs[j] * sv[j] + pe for j in range(BBLK // LANES)]
            for j in range(BBLK // LANES):
                plsc.store_scatter(o, [rov, etv, erv, rows_j[j]], vals[j])
            return carry2

        lax.fori_loop(0, EMBED, e_body, 0, unroll=2)

        pltpu.async_copy(o.at[ro], out5_hbm.at[l, :, wid], sem_o)
        return carry

    lax.fori_loop(0, L, l_body, 0)

    for _ in range(2):
        pltpu.make_async_copy(o.at[0], out5_hbm.at[0, :, 0], sem_o).wait()


@functools.lru_cache(maxsize=None)
def _emb_call():
    return functools.partial(
        pl.kernel,
        out_type=jax.ShapeDtypeStruct((L, EMBED // 8, B // BBLK, 8, BBLK),
                                      jnp.float32),
        mesh=plsc.VectorSubcoreMesh(
            core_axis_name="c", subcore_axis_name="s",
            num_cores=NC, num_subcores=NS),
        scratch_types=[
            pltpu.VMEM((LT, 8, BBLK), jnp.int32),
            pltpu.VMEM((DEPTH, BBLK, EMBED), jnp.float32),
            pltpu.VMEM((2, EMBED // 8, 8, BBLK), jnp.float32),
            pltpu.VMEM((DEPTH, EMBED, LANES), jnp.float32),
            pltpu.SemaphoreType.DMA,
            pltpu.SemaphoreType.DMA,
            pltpu.SemaphoreType.DMA,
        ],
        compiler_params=pltpu.CompilerParams(use_tc_tiling_on_sc=False,
                                             needs_layout_passes=False),
    )(_body)


def kernel(x, table):
    x4 = (x.astype(jnp.int32)
          .reshape(B // BBLK, BBLK, LT, 8)
          .transpose(2, 0, 3, 1))
    pos = _make_pos_embed(MAXLEN, EMBED)[:L]
    ecol = (jnp.arange(EMBED)[:, None] + jnp.arange(LANES)[None, :]) % EMBED
    posb = pos[:, ecol]
    out5 = _emb_call()(x4, table, posb)
    return out5.transpose(2, 4, 0, 1, 3).reshape(B, L, EMBED)
